# Initial kernel scaffold; baseline (speedup 1.0000x reference)
import jax
import jax.numpy as jnp
from jax.experimental import pallas as pl


def kernel(x, edge_index, W1, b1, W2, b2):
    raise NotImplementedError("write your pallas kernel here")


# speedup vs baseline: 12.5590x; 12.5590x over previous
import functools

import jax
import jax.numpy as jnp
from jax import lax
from jax.experimental import pallas as pl
from jax.experimental.pallas import tpu as pltpu
from jax.experimental.pallas import tpu_sc as plsc

N = 10000
E = 320000
D = 128
NC = 2
NS = 16
NW = NC * NS
CH = 128
CHUNKS = -(-E // (NW * CH))
EPW = CHUNKS * CH
EPAD = EPW * NW
NACC = 10112
RPT = NACC // NS
NDEG = 10240
DPT = NDEG // NS

_sc_mesh = plsc.VectorSubcoreMesh(core_axis_name="c", subcore_axis_name="s")


@functools.partial(
    pl.kernel,
    out_type=jax.ShapeDtypeStruct((NC, NDEG), jnp.float32),
    mesh=_sc_mesh,
    scratch_types=[
        pltpu.VMEM((CHUNKS, CH), jnp.int32),
        pltpu.VMEM((CH,), jnp.float32),
        pltpu.VMEM((DPT,), jnp.float32),
        pltpu.VMEM_SHARED((NDEG,), jnp.float32),
    ],
)
def _sc_degree(dst_hbm, out_hbm, dstv, onesv, zv, acc):
    cid = lax.axis_index("c")
    sid = lax.axis_index("s")
    wid = cid * NS + sid
    for i in range(CH // 16):
        onesv[pl.ds(i * 16, 16)] = jnp.ones((16,), jnp.float32)
    for i in range(DPT // 16):
        zv[pl.ds(i * 16, 16)] = jnp.zeros((16,), jnp.float32)
    pltpu.sync_copy(zv, acc.at[pl.ds(sid * DPT, DPT)])
    pltpu.sync_copy(dst_hbm.at[wid], dstv)
    plsc.subcore_barrier()

    @pl.loop(0, CHUNKS)
    def _chunk(j):
        pltpu.sync_copy(onesv, acc.at[dstv.at[j]], add=True)

    plsc.subcore_barrier()
    pltpu.sync_copy(acc.at[pl.ds(sid * DPT, DPT)], out_hbm.at[cid, pl.ds(sid * DPT, DPT)])


@functools.partial(
    pl.kernel,
    out_type=jax.ShapeDtypeStruct((NC, NACC, D), jnp.float32),
    mesh=_sc_mesh,
    scratch_types=[
        pltpu.VMEM((CHUNKS, CH), jnp.int32),
        pltpu.VMEM((CHUNKS, CH), jnp.int32),
        pltpu.VMEM((CH, D), jnp.float32),
        pltpu.VMEM_SHARED((NACC, D), jnp.float32),
        pltpu.SemaphoreType.DMA,
    ],
)
def _sc_scatter(hp_hbm, src_hbm, dst_hbm, zeros_hbm, out_hbm,
                srcv, dstv, rows, acc, sem):
    cid = lax.axis_index("c")
    sid = lax.axis_index("s")
    wid = cid * NS + sid
    pltpu.sync_copy(zeros_hbm.at[pl.ds(sid * RPT, RPT)], acc.at[pl.ds(sid * RPT, RPT)])
    pltpu.sync_copy(src_hbm.at[wid], srcv)
    pltpu.sync_copy(dst_hbm.at[wid], dstv)
    plsc.subcore_barrier()

    @pl.loop(0, CHUNKS)
    def _chunk(j):
        pltpu.async_copy(hp_hbm.at[srcv.at[j]], rows, sem).wait()
        pltpu.sync_copy(rows, acc.at[dstv.at[j]], add=True)

    plsc.subcore_barrier()
    pltpu.sync_copy(acc.at[pl.ds(sid * RPT, RPT)], out_hbm.at[cid, pl.ds(sid * RPT, RPT)])


def _dinv_body(deg_ref, out_ref):
    out_ref[...] = lax.rsqrt(deg_ref[0] + deg_ref[1] + 1.0)


def _mm_scale_body(x_ref, w_ref, dinv_ref, out_ref):
    h = jnp.dot(x_ref[...], w_ref[...], preferred_element_type=jnp.float32)
    out_ref[...] = h * dinv_ref[...]


def _mid_body(s_ref, hp_ref, dinv_ref, b_ref, w_ref, out_ref):
    h1 = (s_ref[0] + s_ref[1] + hp_ref[...]) * dinv_ref[...] + b_ref[...]
    h1 = jnp.maximum(h1, 0.0)
    out_ref[...] = jnp.dot(h1, w_ref[...], preferred_element_type=jnp.float32) * dinv_ref[...]


def _final_body(s_ref, hp_ref, dinv_ref, b_ref, out_ref):
    out_ref[...] = (s_ref[0] + s_ref[1] + hp_ref[...]) * dinv_ref[...] + b_ref[...]


_RB = 400

_row_spec = pl.BlockSpec((_RB, D), lambda i: (i, 0))
_dinv_spec = pl.BlockSpec((_RB, 1), lambda i: (i, 0))
_s_spec = pl.BlockSpec((NC, _RB, D), lambda i: (0, i, 0))
_w_spec = pl.BlockSpec((D, D), lambda i: (0, 0))
_b_spec = pl.BlockSpec((1, D), lambda i: (0, 0))
_out_struct = jax.ShapeDtypeStruct((N, D), jnp.float32)

_dinv_tc = pl.pallas_call(
    _dinv_body,
    out_shape=jax.ShapeDtypeStruct((NDEG,), jnp.float32),
)

_mm_scale = pl.pallas_call(
    _mm_scale_body,
    grid=(N // _RB,),
    in_specs=[_row_spec, _w_spec, _dinv_spec],
    out_specs=_row_spec,
    out_shape=_out_struct,
)

_mid = pl.pallas_call(
    _mid_body,
    grid=(N // _RB,),
    in_specs=[_s_spec, _row_spec, _dinv_spec, _b_spec, _w_spec],
    out_specs=_row_spec,
    out_shape=_out_struct,
)

_final = pl.pallas_call(
    _final_body,
    grid=(N // _RB,),
    in_specs=[_s_spec, _row_spec, _dinv_spec, _b_spec],
    out_specs=_row_spec,
    out_shape=_out_struct,
)


def kernel(x, edge_index, W1, b1, W2, b2):
    src = edge_index[0].astype(jnp.int32)
    dst = edge_index[1].astype(jnp.int32)
    src_p = jnp.concatenate([src, jnp.zeros((EPAD - E,), jnp.int32)]).reshape(NW, CHUNKS, CH)
    dst_p = jnp.concatenate([dst, jnp.full((EPAD - E,), N, jnp.int32)]).reshape(NW, CHUNKS, CH)
    zeros = jnp.zeros((NACC, D), jnp.float32)

    deg_part = _sc_degree(dst_p)
    dinv = _dinv_tc(deg_part)
    dinv2d = dinv[:N, None]
    b1r = b1.reshape(1, D)
    b2r = b2.reshape(1, D)

    hp1 = _mm_scale(x, W1, dinv2d)
    s1 = _sc_scatter(hp1, src_p, dst_p, zeros)
    hp2 = _mid(s1, hp1, dinv2d, b1r, W2)
    s2 = _sc_scatter(hp2, src_p, dst_p, zeros)
    return _final(s2, hp2, dinv2d, b2r)
